# baseline (speedup 1.0000x reference)
import jax
import jax.numpy as jnp
from jax import lax
from jax.experimental import pallas as pl

HIDDEN = 500
N_NODES = 31


def _fused_kernel(x_ref, w_ref, b_ref, no_ref, out_ref):
    x = x_ref[...]
    w = w_ref[...]
    b = b_ref[...]
    no2 = no_ref[0]

    acc = lax.dot_general(
        x, w,
        dimension_numbers=(((1,), (1,)), ((), ())),
        preferred_element_type=jnp.float32,
    ) + b

    rows, cols = acc.shape
    n2 = lax.broadcasted_iota(jnp.int32, no2.shape, 1)
    max_order = jnp.max(no2)
    cond = (n2 != 0) & (no2 < 5) & (no2 < max_order)
    target = jnp.where(cond, (3 * no2 + (n2 + 2) % 3).astype(jnp.float32),
                       -1.0)
    tcol = jnp.concatenate(
        [lax.transpose(target[0:1, :], (1, 0)),
         lax.transpose(target[1:2, :], (1, 0))], axis=0)
    h_f = lax.broadcasted_iota(jnp.int32, (rows, cols), 1).astype(jnp.float32)
    out_ref[...] = acc + (h_f == tcol).astype(jnp.float32)


def kernel(forest, adjacency, node_order, edge_order, W, b):
    batch, n_agents, n_nodes, feat = forest.shape
    rows = batch * n_agents * n_nodes
    x = forest.reshape(rows, feat)
    b2 = b.reshape(1, HIDDEN)

    out = pl.pallas_call(
        _fused_kernel,
        out_shape=jax.ShapeDtypeStruct((rows, HIDDEN), jnp.float32),
    )(x, W, b2, node_order.astype(jnp.int32))
    return out.reshape(batch, n_agents, n_nodes, HIDDEN)
